# baseline (speedup 1.0000x reference)
import functools
import math

import jax
import jax.numpy as jnp
from jax import lax
from jax.experimental import pallas as pl
from jax.experimental.pallas import tpu as pltpu
from jax.experimental.pallas import tpu_sc as plsc

EMBED = 64
SCALE = math.sqrt(EMBED)

NC = 2
NS = 16
NW = NC * NS

CHUNK = 128
NBUF = 4
LANES = 16
VPR = EMBED // LANES


def _body(nchunk, idx_hbm, table_hbm, out_hbm, idx_v,
          i0, i1, i2, i3, o0, o1, o2, o3,
          g0, g1, g2, g3, s0, s1, s2, s3):
  ib = (i0, i1, i2, i3)
  ob = (o0, o1, o2, o3)
  gs = (g0, g1, g2, g3)
  os_ = (s0, s1, s2, s3)

  wid = lax.axis_index("s") * NC + lax.axis_index("c")

  pltpu.sync_copy(idx_hbm.at[wid], idx_v)

  for b in range(NBUF):
    pltpu.async_copy(table_hbm.at[idx_v.at[b]], ib[b], gs[b])

  @pl.loop(0, nchunk, step=NBUF)
  def _(j0):
    for b in range(NBUF):
      j = j0 + b
      pltpu.make_async_copy(table_hbm.at[idx_v.at[j]], ib[b], gs[b]).wait()

      @pl.when(j0 > 0)
      def _():
        pltpu.make_async_copy(ob[b], out_hbm.at[wid, j], os_[b]).wait()

      @pl.loop(0, CHUNK, unroll=8)
      def _(r):
        for c in range(VPR):
          ob[b][r, pl.ds(c * LANES, LANES)] = (
              ib[b][r, pl.ds(c * LANES, LANES)] * SCALE)

      @pl.when(j0 + 2 * NBUF <= nchunk)
      def _():
        pltpu.async_copy(table_hbm.at[idx_v.at[j + NBUF]], ib[b], gs[b])

      pltpu.async_copy(ob[b], out_hbm.at[wid, j], os_[b])

  for b in range(NBUF):
    pltpu.make_async_copy(ob[b], out_hbm.at[wid, 0], os_[b]).wait()


@functools.partial(jax.jit, static_argnames=("nchunk",))
def _embed_sc(idx, table, nchunk):
  mesh = plsc.VectorSubcoreMesh(core_axis_name="c", subcore_axis_name="s")
  f = pl.kernel(
      functools.partial(_body, nchunk),
      out_type=jax.ShapeDtypeStruct((NW, nchunk, CHUNK, EMBED), jnp.float32),
      mesh=mesh,
      compiler_params=pltpu.CompilerParams(use_tc_tiling_on_sc=False),
      scratch_types=(
          [pltpu.VMEM((nchunk, CHUNK), jnp.int32)]
          + [pltpu.VMEM((CHUNK, EMBED), jnp.float32)] * (2 * NBUF)
          + [pltpu.SemaphoreType.DMA] * (2 * NBUF)
      ),
  )
  return f(idx, table)


def kernel(tokens, table):
  ntok = tokens.shape[0] * tokens.shape[1]
  nchunk = ntok // (NW * CHUNK)
  idx = tokens.astype(jnp.int32).reshape(NW, nchunk, CHUNK)
  out = _embed_sc(idx, table, nchunk)
  return out.reshape(tokens.shape[0], tokens.shape[1], EMBED)
